# baseline (speedup 1.0000x reference)
import functools
import jax
import jax.numpy as jnp
from jax import lax
from jax.experimental import pallas as pl
from jax.experimental.pallas import tpu as pltpu
from jax.experimental.pallas import tpu_sc as plsc

T = 2048
D = 2048
HQ = 16
HKV = 8
HD = 128
E = 8
K = 2
F = 2048
C = 1024
CP = 1032
EPS = 1e-5
BASE = 150000.0
ALPHA = 1.702
LIMIT = 7.0


def _rmsmat_body(x_ref, w_ref, b_ref, g_ref, o_ref):
    x = x_ref[...]
    h = x * lax.rsqrt(jnp.mean(x * x, axis=-1, keepdims=True) + EPS) * g_ref[...]
    o_ref[...] = jnp.dot(h, w_ref[...], preferred_element_type=jnp.float32) + b_ref[...]


def _rms_matmul(x, w, b, g, bt=256, bn=1024):
    n = w.shape[1]
    return pl.pallas_call(
        _rmsmat_body,
        grid=(T // bt, n // bn),
        in_specs=[
            pl.BlockSpec((bt, D), lambda t, j: (t, 0)),
            pl.BlockSpec((D, bn), lambda t, j: (0, j)),
            pl.BlockSpec((1, bn), lambda t, j: (0, j)),
            pl.BlockSpec((1, D), lambda t, j: (0, 0)),
        ],
        out_specs=pl.BlockSpec((bt, bn), lambda t, j: (t, j)),
        out_shape=jax.ShapeDtypeStruct((T, n), jnp.float32),
    )(x, w, b.reshape(1, n), g.reshape(1, D))


def _rope_body(x1_ref, x2_ref, t_ref, inv_ref, o1_ref, o2_ref, *, nh):
    ang = t_ref[...] * inv_ref[...]
    c = jnp.cos(ang)
    s = jnp.sin(ang)
    cf = jnp.concatenate([c] * nh, axis=1)
    sf = jnp.concatenate([s] * nh, axis=1)
    x1 = x1_ref[...]
    x2 = x2_ref[...]
    o1_ref[...] = x1 * cf - x2 * sf
    o2_ref[...] = x2 * cf + x1 * sf


def _rope(x, pos_f, inv, nh, bt=256):
    half = HD // 2
    w = nh * half
    x1 = x[:, :, :half].reshape(T, w)
    x2 = x[:, :, half:].reshape(T, w)
    o1, o2 = pl.pallas_call(
        functools.partial(_rope_body, nh=nh),
        grid=(T // bt,),
        in_specs=[
            pl.BlockSpec((bt, w), lambda r: (r, 0)),
            pl.BlockSpec((bt, w), lambda r: (r, 0)),
            pl.BlockSpec((bt, 1), lambda r: (r, 0)),
            pl.BlockSpec((1, half), lambda r: (0, 0)),
        ],
        out_specs=[
            pl.BlockSpec((bt, w), lambda r: (r, 0)),
            pl.BlockSpec((bt, w), lambda r: (r, 0)),
        ],
        out_shape=[
            jax.ShapeDtypeStruct((T, w), jnp.float32),
            jax.ShapeDtypeStruct((T, w), jnp.float32),
        ],
    )(x1, x2, pos_f, inv)
    o1 = o1.reshape(T, nh, half)
    o2 = o2.reshape(T, nh, half)
    return jnp.concatenate([o1, o2], axis=2)


def _attn_body(q_ref, k_ref, v_ref, sink_ref, o_ref, *, bq):
    h = pl.program_id(0)
    tq = pl.program_id(1)
    q = q_ref[...]
    k = k_ref[...]
    s = lax.dot_general(q, k, (((1,), (1,)), ((), ())),
                        preferred_element_type=jnp.float32) * (HD ** -0.5)
    row = tq * bq + lax.broadcasted_iota(jnp.int32, s.shape, 0)
    col = lax.broadcasted_iota(jnp.int32, s.shape, 1)
    s = jnp.where(col <= row, s, -1e30)
    sink = sink_ref[h, 0]
    m = jnp.maximum(jnp.max(s, axis=-1, keepdims=True), sink)
    p = jnp.exp(s - m)
    l = jnp.sum(p, axis=-1, keepdims=True) + jnp.exp(sink - m)
    o = jnp.dot(p, v_ref[...], preferred_element_type=jnp.float32)
    o_ref[...] = o / l


def _attention(q, k, v, sinks, bq=256):
    return pl.pallas_call(
        functools.partial(_attn_body, bq=bq),
        grid=(HQ, T // bq),
        in_specs=[
            pl.BlockSpec((bq, HD), lambda h, tq: (tq, h)),
            pl.BlockSpec((T, HD), lambda h, tq: (0, h // (HQ // HKV))),
            pl.BlockSpec((T, HD), lambda h, tq: (0, h // (HQ // HKV))),
            pl.BlockSpec((HQ, 1), lambda h, tq: (0, 0)),
        ],
        out_specs=pl.BlockSpec((bq, HD), lambda h, tq: (tq, h)),
        out_shape=jax.ShapeDtypeStruct((T, HQ * HD), jnp.float32),
    )(q, k, v, sinks.reshape(HQ, 1))


def _oproj_body(a_ref, w_ref, b_ref, r_ref, o_ref):
    o_ref[...] = (jnp.dot(a_ref[...], w_ref[...], preferred_element_type=jnp.float32)
                  + b_ref[...] + r_ref[...])


def _oproj_residual(a, w, b, res, bt=256, bn=1024):
    return pl.pallas_call(
        _oproj_body,
        grid=(T // bt, D // bn),
        in_specs=[
            pl.BlockSpec((bt, D), lambda t, j: (t, 0)),
            pl.BlockSpec((D, bn), lambda t, j: (0, j)),
            pl.BlockSpec((1, bn), lambda t, j: (0, j)),
            pl.BlockSpec((bt, bn), lambda t, j: (t, j)),
        ],
        out_specs=pl.BlockSpec((bt, bn), lambda t, j: (t, j)),
        out_shape=jax.ShapeDtypeStruct((T, D), jnp.float32),
    )(a, w, b.reshape(1, D), res)


def _rms2_body(x_ref, g_ref, wr_ref, br_ref, h_ref, lg_ref):
    x = x_ref[...]
    h = x * lax.rsqrt(jnp.mean(x * x, axis=-1, keepdims=True) + EPS) * g_ref[...]
    h_ref[...] = h
    lg_ref[...] = jnp.dot(h, wr_ref[...], preferred_element_type=jnp.float32) + br_ref[...]


def _rms2_logits(x, g, wr, br, bt=256):
    return pl.pallas_call(
        _rms2_body,
        grid=(T // bt,),
        in_specs=[
            pl.BlockSpec((bt, D), lambda t: (t, 0)),
            pl.BlockSpec((1, D), lambda t: (0, 0)),
            pl.BlockSpec((D, E), lambda t: (0, 0)),
            pl.BlockSpec((1, E), lambda t: (0, 0)),
        ],
        out_specs=[
            pl.BlockSpec((bt, D), lambda t: (t, 0)),
            pl.BlockSpec((bt, E), lambda t: (t, 0)),
        ],
        out_shape=[
            jax.ShapeDtypeStruct((T, D), jnp.float32),
            jax.ShapeDtypeStruct((T, E), jnp.float32),
        ],
    )(x, g.reshape(1, D), wr, br.reshape(1, E))


def _route_body(lg_ref, dest_ref, csrc_ref, cw_ref, cnt_ref):
    lg = lg_ref[...]
    ie = lax.broadcasted_iota(jnp.int32, lg.shape, 1)
    m1 = jnp.max(lg, axis=1, keepdims=True)
    a1 = jnp.min(jnp.where(lg == m1, ie, E), axis=1, keepdims=True)
    lg2 = jnp.where(ie == a1, -jnp.inf, lg)
    m2 = jnp.max(lg2, axis=1, keepdims=True)
    a2 = jnp.min(jnp.where(lg2 == m2, ie, E), axis=1, keepdims=True)
    w1 = 1.0 / (1.0 + jnp.exp(m2 - m1))
    w2 = 1.0 - w1
    oh1 = (ie == a1).astype(jnp.int32)
    oh2 = (ie == a2).astype(jnp.int32)
    ohs = oh1 + oh2
    c = ohs
    s = 1
    while s < T:
        rolled = pltpu.roll(c, s, 0)
        ir = lax.broadcasted_iota(jnp.int32, c.shape, 0)
        c = c + jnp.where(ir < s, 0, rolled)
        s *= 2
    p = c - ohs
    slot1 = jnp.sum(p * oh1, axis=1, keepdims=True)
    slot2 = jnp.sum(p * oh2, axis=1, keepdims=True)
    keep1 = slot1 < C
    keep2 = slot2 < C
    d1 = a1 * CP + jnp.where(keep1, slot1, C)
    d2 = a2 * CP + jnp.where(keep2, slot2, C)
    dest_ref[...] = jnp.concatenate([d1, d2], axis=1)
    c1 = a1 * C + jnp.minimum(slot1, C - 1)
    c2 = a2 * C + jnp.minimum(slot2, C - 1)
    csrc_ref[...] = jnp.concatenate([c1, c2], axis=1)
    cw1 = jnp.where(keep1, w1, 0.0)
    cw2 = jnp.where(keep2, w2, 0.0)
    cw_ref[...] = jnp.concatenate([cw1, cw2], axis=1)
    cnt_ref[...] = c[T - 1:T, :]


def _route(logits):
    return pl.pallas_call(
        _route_body,
        out_shape=[
            jax.ShapeDtypeStruct((T, K), jnp.int32),
            jax.ShapeDtypeStruct((T, K), jnp.int32),
            jax.ShapeDtypeStruct((T, K), jnp.float32),
            jax.ShapeDtypeStruct((1, E), jnp.int32),
        ],
    )(logits)


_NC = 2
_NS = 16
_NW = _NC * _NS
_CHUNK = 32
_EPW = (T * K) // _NW


def _sc_dispatch_body(h2_hbm, dest_hbm, buf_hbm, dvec, tokvec, rows, sem):
    wid = lax.axis_index("s") * _NC + lax.axis_index("c")
    for cchunk in range(_EPW // _CHUNK):
        base = wid * _EPW + cchunk * _CHUNK
        pltpu.sync_copy(dest_hbm.at[pl.ds(base, _CHUNK)], dvec)
        for h in range(_CHUNK // 16):
            j = base + h * 16 + lax.iota(jnp.int32, 16)
            tokvec[pl.ds(h * 16, 16)] = lax.shift_right_logical(j, 1)
        pltpu.async_copy(h2_hbm.at[tokvec], rows, sem).wait()
        pltpu.async_copy(rows, buf_hbm.at[dvec], sem).wait()


def _sc_dispatch(h2, dest_flat):
    mesh = plsc.VectorSubcoreMesh(core_axis_name="c", subcore_axis_name="s")
    fn = functools.partial(
        pl.kernel,
        mesh=mesh,
        out_type=jax.ShapeDtypeStruct((E * CP, D), jnp.float32),
        scratch_types=[
            pltpu.VMEM((_CHUNK,), jnp.int32),
            pltpu.VMEM((_CHUNK,), jnp.int32),
            pltpu.VMEM((_CHUNK, D), jnp.float32),
            pltpu.SemaphoreType.DMA,
        ],
    )(_sc_dispatch_body)
    return fn(h2, dest_flat)


def _sc_gather_body(y_hbm, csrc_hbm, out_hbm, cvec, rows, sem):
    wid = lax.axis_index("s") * _NC + lax.axis_index("c")
    for cchunk in range(_EPW // _CHUNK):
        base = wid * _EPW + cchunk * _CHUNK
        pltpu.sync_copy(csrc_hbm.at[pl.ds(base, _CHUNK)], cvec)
        pltpu.async_copy(y_hbm.at[cvec], rows, sem).wait()
        pltpu.sync_copy(rows, out_hbm.at[pl.ds(base, _CHUNK)])


def _sc_gather(y, csrc_flat):
    mesh = plsc.VectorSubcoreMesh(core_axis_name="c", subcore_axis_name="s")
    fn = functools.partial(
        pl.kernel,
        mesh=mesh,
        out_type=jax.ShapeDtypeStruct((T * K, D), jnp.float32),
        scratch_types=[
            pltpu.VMEM((_CHUNK,), jnp.int32),
            pltpu.VMEM((_CHUNK, D), jnp.float32),
            pltpu.SemaphoreType.DMA,
        ],
    )(_sc_gather_body)
    return fn(y, csrc_flat)


def _expert_body(cnt_ref, x_ref, wg_ref, wu_ref, bg_ref, bu_ref, wd_ref, bd_ref,
                 o_ref, *, bc):
    e = pl.program_id(0)
    cb = pl.program_id(1)
    fb = pl.program_id(2)

    @pl.when(cb * bc < cnt_ref[0, e])
    def _():
        x = x_ref[0]
        g = jnp.dot(x, wg_ref[0], preferred_element_type=jnp.float32) + bg_ref[0]
        u = jnp.dot(x, wu_ref[0], preferred_element_type=jnp.float32) + bu_ref[0]
        gate = jnp.minimum(g, LIMIT)
        up = jnp.clip(u, -LIMIT, LIMIT)
        glu = gate * jax.nn.sigmoid(ALPHA * gate)
        act = (up + 1.0) * glu
        part = jnp.dot(act, wd_ref[0], preferred_element_type=jnp.float32)

        @pl.when(fb == 0)
        def _():
            o_ref[0] = part + bd_ref[0]

        @pl.when(fb != 0)
        def _():
            o_ref[0] = o_ref[0] + part


def _experts(counts, buf, wg, wu, bg, bu, wd, bd, bc=256, bf=512):
    return pl.pallas_call(
        functools.partial(_expert_body, bc=bc),
        grid=(E, C // bc, F // bf),
        in_specs=[
            pl.BlockSpec(memory_space=pltpu.SMEM),
            pl.BlockSpec((1, bc, D), lambda e, cb, fb: (e, cb, 0)),
            pl.BlockSpec((1, D, bf), lambda e, cb, fb: (e, 0, fb)),
            pl.BlockSpec((1, D, bf), lambda e, cb, fb: (e, 0, fb)),
            pl.BlockSpec((1, 1, bf), lambda e, cb, fb: (e, 0, fb)),
            pl.BlockSpec((1, 1, bf), lambda e, cb, fb: (e, 0, fb)),
            pl.BlockSpec((1, bf, D), lambda e, cb, fb: (e, fb, 0)),
            pl.BlockSpec((1, 1, D), lambda e, cb, fb: (e, 0, 0)),
        ],
        out_specs=pl.BlockSpec((1, bc, D), lambda e, cb, fb: (e, cb, 0)),
        out_shape=jax.ShapeDtypeStruct((E, C, D), jnp.float32),
    )(counts, buf, wg, wu, bg, bu, wd, bd)


def _combine_body(x_ref, g_ref, cw_ref, o_ref):
    cw1 = cw_ref[:, 0:1]
    cw2 = cw_ref[:, 1:2]
    g = g_ref[...]
    o_ref[...] = x_ref[...] + cw1 * g[:, :D] + cw2 * g[:, D:]


def _combine(x, gath2, cw, bt=256):
    return pl.pallas_call(
        _combine_body,
        grid=(T // bt,),
        in_specs=[
            pl.BlockSpec((bt, D), lambda t: (t, 0)),
            pl.BlockSpec((bt, K * D), lambda t: (t, 0)),
            pl.BlockSpec((bt, K), lambda t: (t, 0)),
        ],
        out_specs=pl.BlockSpec((bt, D), lambda t: (t, 0)),
        out_shape=jax.ShapeDtypeStruct((T, D), jnp.float32),
    )(x, gath2, cw)


def kernel(hidden_states, positions, ln1_w, w_qkv, b_qkv, sinks, w_o, b_o,
           ln2_w, w_router, b_router, w_gate_up, b_gate_up, w_down, b_down):
    half = HD // 2
    qkv = _rms_matmul(hidden_states, w_qkv, b_qkv, ln1_w)
    q = qkv[:, :HQ * HD].reshape(T, HQ, HD)
    k = qkv[:, HQ * HD:(HQ + HKV) * HD].reshape(T, HKV, HD)
    v = qkv[:, (HQ + HKV) * HD:]

    pos_f = positions.astype(jnp.float32).reshape(T, 1)
    inv = (1.0 / (BASE ** (jnp.arange(half, dtype=jnp.float32) / half))).reshape(1, half)
    q = _rope(q, pos_f, inv, HQ).reshape(T, HQ * HD)
    k = _rope(k, pos_f, inv, HKV).reshape(T, HKV * HD)

    a = _attention(q, k, v, sinks)
    x = _oproj_residual(a, w_o, b_o, hidden_states)
    h2, logits = _rms2_logits(x, ln2_w, w_router, b_router)

    dest, csrc, cw, counts = _route(logits)
    buf = _sc_dispatch(h2, dest.reshape(T * K)).reshape(E, CP, D)

    wg = w_gate_up[:, :, :F]
    wu = w_gate_up[:, :, F:]
    bg = b_gate_up[:, :F].reshape(E, 1, F)
    bu = b_gate_up[:, F:].reshape(E, 1, F)
    y = _experts(counts, buf, wg, wu, bg, bu, w_down, b_down.reshape(E, 1, D))

    gath = _sc_gather(y.reshape(E * C, D), csrc.reshape(T * K))
    return _combine(x, gath.reshape(T, K * D), cw)


# speedup vs baseline: 1.1694x; 1.1694x over previous
import functools
import jax
import jax.numpy as jnp
from jax import lax
from jax.experimental import pallas as pl
from jax.experimental.pallas import tpu as pltpu
from jax.experimental.pallas import tpu_sc as plsc

T = 2048
D = 2048
HQ = 16
HKV = 8
HD = 128
E = 8
K = 2
F = 2048
C = 1024
CP = 1032
EPS = 1e-5
BASE = 150000.0
ALPHA = 1.702
LIMIT = 7.0


def _rmsmat_body(x_ref, w_ref, b_ref, g_ref, o_ref):
    x = x_ref[...]
    h = x * lax.rsqrt(jnp.mean(x * x, axis=-1, keepdims=True) + EPS) * g_ref[...]
    o_ref[...] = jnp.dot(h, w_ref[...], preferred_element_type=jnp.float32) + b_ref[...]


def _rms_matmul(x, w, b, g, bt=256, bn=1024):
    n = w.shape[1]
    return pl.pallas_call(
        _rmsmat_body,
        grid=(T // bt, n // bn),
        in_specs=[
            pl.BlockSpec((bt, D), lambda t, j: (t, 0)),
            pl.BlockSpec((D, bn), lambda t, j: (0, j)),
            pl.BlockSpec((1, bn), lambda t, j: (0, j)),
            pl.BlockSpec((1, D), lambda t, j: (0, 0)),
        ],
        out_specs=pl.BlockSpec((bt, bn), lambda t, j: (t, j)),
        out_shape=jax.ShapeDtypeStruct((T, n), jnp.float32),
    )(x, w, b.reshape(1, n), g.reshape(1, D))


def _rope_body(x1_ref, x2_ref, t_ref, inv_ref, o1_ref, o2_ref, *, nh):
    ang = t_ref[...] * inv_ref[...]
    c = jnp.cos(ang)
    s = jnp.sin(ang)
    cf = jnp.concatenate([c] * nh, axis=1)
    sf = jnp.concatenate([s] * nh, axis=1)
    x1 = x1_ref[...]
    x2 = x2_ref[...]
    o1_ref[...] = x1 * cf - x2 * sf
    o2_ref[...] = x2 * cf + x1 * sf


def _rope(x, pos_f, inv, nh, bt=256):
    half = HD // 2
    w = nh * half
    x1 = x[:, :, :half].reshape(T, w)
    x2 = x[:, :, half:].reshape(T, w)
    o1, o2 = pl.pallas_call(
        functools.partial(_rope_body, nh=nh),
        grid=(T // bt,),
        in_specs=[
            pl.BlockSpec((bt, w), lambda r: (r, 0)),
            pl.BlockSpec((bt, w), lambda r: (r, 0)),
            pl.BlockSpec((bt, 1), lambda r: (r, 0)),
            pl.BlockSpec((1, half), lambda r: (0, 0)),
        ],
        out_specs=[
            pl.BlockSpec((bt, w), lambda r: (r, 0)),
            pl.BlockSpec((bt, w), lambda r: (r, 0)),
        ],
        out_shape=[
            jax.ShapeDtypeStruct((T, w), jnp.float32),
            jax.ShapeDtypeStruct((T, w), jnp.float32),
        ],
    )(x1, x2, pos_f, inv)
    o1 = o1.reshape(T, nh, half)
    o2 = o2.reshape(T, nh, half)
    return jnp.concatenate([o1, o2], axis=2)


def _attn_body(q_ref, k_ref, v_ref, sink_ref, o_ref, *, bq):
    h = pl.program_id(0)
    tq = pl.program_id(1)
    q = q_ref[...]
    k = k_ref[...]
    s = lax.dot_general(q, k, (((1,), (1,)), ((), ())),
                        preferred_element_type=jnp.float32) * (HD ** -0.5)
    row = tq * bq + lax.broadcasted_iota(jnp.int32, s.shape, 0)
    col = lax.broadcasted_iota(jnp.int32, s.shape, 1)
    s = jnp.where(col <= row, s, -1e30)
    sink = sink_ref[h, 0]
    m = jnp.maximum(jnp.max(s, axis=-1, keepdims=True), sink)
    p = jnp.exp(s - m)
    l = jnp.sum(p, axis=-1, keepdims=True) + jnp.exp(sink - m)
    o = jnp.dot(p, v_ref[...], preferred_element_type=jnp.float32)
    o_ref[...] = o / l


def _attention(q, k, v, sinks, bq=256):
    return pl.pallas_call(
        functools.partial(_attn_body, bq=bq),
        grid=(HQ, T // bq),
        in_specs=[
            pl.BlockSpec((bq, HD), lambda h, tq: (tq, h)),
            pl.BlockSpec((T, HD), lambda h, tq: (0, h // (HQ // HKV))),
            pl.BlockSpec((T, HD), lambda h, tq: (0, h // (HQ // HKV))),
            pl.BlockSpec((HQ, 1), lambda h, tq: (0, 0)),
        ],
        out_specs=pl.BlockSpec((bq, HD), lambda h, tq: (tq, h)),
        out_shape=jax.ShapeDtypeStruct((T, HQ * HD), jnp.float32),
    )(q, k, v, sinks.reshape(HQ, 1))


def _oproj_body(a_ref, w_ref, b_ref, r_ref, o_ref):
    o_ref[...] = (jnp.dot(a_ref[...], w_ref[...], preferred_element_type=jnp.float32)
                  + b_ref[...] + r_ref[...])


def _oproj_residual(a, w, b, res, bt=256, bn=1024):
    return pl.pallas_call(
        _oproj_body,
        grid=(T // bt, D // bn),
        in_specs=[
            pl.BlockSpec((bt, D), lambda t, j: (t, 0)),
            pl.BlockSpec((D, bn), lambda t, j: (0, j)),
            pl.BlockSpec((1, bn), lambda t, j: (0, j)),
            pl.BlockSpec((bt, bn), lambda t, j: (t, j)),
        ],
        out_specs=pl.BlockSpec((bt, bn), lambda t, j: (t, j)),
        out_shape=jax.ShapeDtypeStruct((T, D), jnp.float32),
    )(a, w, b.reshape(1, D), res)


def _rms2_body(x_ref, g_ref, wr_ref, br_ref, h_ref, lg_ref):
    x = x_ref[...]
    h = x * lax.rsqrt(jnp.mean(x * x, axis=-1, keepdims=True) + EPS) * g_ref[...]
    h_ref[...] = h
    lg_ref[...] = jnp.dot(h, wr_ref[...], preferred_element_type=jnp.float32) + br_ref[...]


def _rms2_logits(x, g, wr, br, bt=256):
    return pl.pallas_call(
        _rms2_body,
        grid=(T // bt,),
        in_specs=[
            pl.BlockSpec((bt, D), lambda t: (t, 0)),
            pl.BlockSpec((1, D), lambda t: (0, 0)),
            pl.BlockSpec((D, E), lambda t: (0, 0)),
            pl.BlockSpec((1, E), lambda t: (0, 0)),
        ],
        out_specs=[
            pl.BlockSpec((bt, D), lambda t: (t, 0)),
            pl.BlockSpec((bt, E), lambda t: (t, 0)),
        ],
        out_shape=[
            jax.ShapeDtypeStruct((T, D), jnp.float32),
            jax.ShapeDtypeStruct((T, E), jnp.float32),
        ],
    )(x, g.reshape(1, D), wr, br.reshape(1, E))


def _route_body(lg_ref, dest_ref, csrc_ref, cw_ref, cnt_ref):
    lg = lg_ref[...]
    ie = lax.broadcasted_iota(jnp.int32, lg.shape, 1)
    m1 = jnp.max(lg, axis=1, keepdims=True)
    a1 = jnp.min(jnp.where(lg == m1, ie, E), axis=1, keepdims=True)
    lg2 = jnp.where(ie == a1, -jnp.inf, lg)
    m2 = jnp.max(lg2, axis=1, keepdims=True)
    a2 = jnp.min(jnp.where(lg2 == m2, ie, E), axis=1, keepdims=True)
    w1 = 1.0 / (1.0 + jnp.exp(m2 - m1))
    w2 = 1.0 - w1
    oh1 = (ie == a1).astype(jnp.int32)
    oh2 = (ie == a2).astype(jnp.int32)
    ohs = oh1 + oh2
    c = ohs
    s = 1
    while s < T:
        rolled = pltpu.roll(c, s, 0)
        ir = lax.broadcasted_iota(jnp.int32, c.shape, 0)
        c = c + jnp.where(ir < s, 0, rolled)
        s *= 2
    p = c - ohs
    slot1 = jnp.sum(p * oh1, axis=1, keepdims=True)
    slot2 = jnp.sum(p * oh2, axis=1, keepdims=True)
    keep1 = slot1 < C
    keep2 = slot2 < C
    d1 = a1 * CP + jnp.where(keep1, slot1, C)
    d2 = a2 * CP + jnp.where(keep2, slot2, C)
    dest_ref[...] = jnp.concatenate([d1, d2], axis=1)
    c1 = a1 * C + jnp.minimum(slot1, C - 1)
    c2 = a2 * C + jnp.minimum(slot2, C - 1)
    csrc_ref[...] = jnp.concatenate([c1, c2], axis=1)
    cw1 = jnp.where(keep1, w1, 0.0)
    cw2 = jnp.where(keep2, w2, 0.0)
    cw_ref[...] = jnp.concatenate([cw1, cw2], axis=1)
    cnt_ref[...] = c[T - 1:T, :]


def _route(logits):
    return pl.pallas_call(
        _route_body,
        out_shape=[
            jax.ShapeDtypeStruct((T, K), jnp.int32),
            jax.ShapeDtypeStruct((T, K), jnp.int32),
            jax.ShapeDtypeStruct((T, K), jnp.float32),
            jax.ShapeDtypeStruct((1, E), jnp.int32),
        ],
    )(logits)


_NC = 2
_NS = 16
_NW = _NC * _NS
_CHUNK = 32
_EPW = (T * K) // _NW


def _sc_dispatch_body(h2_hbm, dest_hbm, buf_hbm, dvec, tokvec, rows, sem):
    wid = lax.axis_index("s") * _NC + lax.axis_index("c")
    for cchunk in range(_EPW // _CHUNK):
        base = wid * _EPW + cchunk * _CHUNK
        pltpu.sync_copy(dest_hbm.at[pl.ds(base, _CHUNK)], dvec)
        for h in range(_CHUNK // 16):
            j = base + h * 16 + lax.iota(jnp.int32, 16)
            tokvec[pl.ds(h * 16, 16)] = lax.shift_right_logical(j, 1)
        pltpu.async_copy(h2_hbm.at[tokvec], rows, sem).wait()
        pltpu.async_copy(rows, buf_hbm.at[dvec], sem).wait()


def _sc_dispatch(h2, dest_flat):
    mesh = plsc.VectorSubcoreMesh(core_axis_name="c", subcore_axis_name="s")
    fn = functools.partial(
        pl.kernel,
        mesh=mesh,
        out_type=jax.ShapeDtypeStruct((E * CP, D), jnp.float32),
        scratch_types=[
            pltpu.VMEM((_CHUNK,), jnp.int32),
            pltpu.VMEM((_CHUNK,), jnp.int32),
            pltpu.VMEM((_CHUNK, D), jnp.float32),
            pltpu.SemaphoreType.DMA,
        ],
    )(_sc_dispatch_body)
    return fn(h2, dest_flat)


def _sc_gather_body(y_hbm, csrc_hbm, out_hbm, cvec, rows, sem):
    wid = lax.axis_index("s") * _NC + lax.axis_index("c")
    for cchunk in range(_EPW // _CHUNK):
        base = wid * _EPW + cchunk * _CHUNK
        pltpu.sync_copy(csrc_hbm.at[pl.ds(base, _CHUNK)], cvec)
        pltpu.async_copy(y_hbm.at[cvec], rows, sem).wait()
        pltpu.sync_copy(rows, out_hbm.at[pl.ds(base, _CHUNK)])


def _sc_gather(y, csrc_flat):
    mesh = plsc.VectorSubcoreMesh(core_axis_name="c", subcore_axis_name="s")
    fn = functools.partial(
        pl.kernel,
        mesh=mesh,
        out_type=jax.ShapeDtypeStruct((T * K, D), jnp.float32),
        scratch_types=[
            pltpu.VMEM((_CHUNK,), jnp.int32),
            pltpu.VMEM((_CHUNK, D), jnp.float32),
            pltpu.SemaphoreType.DMA,
        ],
    )(_sc_gather_body)
    return fn(y, csrc_flat)


def _expert_body(cnt_ref, x_ref, wg_ref, wu_ref, bg_ref, bu_ref, wd_ref, bd_ref,
                 o_ref, *, bc):
    e = pl.program_id(0)
    cb = pl.program_id(1)
    fb = pl.program_id(2)

    @pl.when(cb * bc < cnt_ref[0, e])
    def _():
        x = x_ref[0]
        g = jnp.dot(x, wg_ref[0], preferred_element_type=jnp.float32) + bg_ref[0]
        u = jnp.dot(x, wu_ref[0], preferred_element_type=jnp.float32) + bu_ref[0]
        gate = jnp.minimum(g, LIMIT)
        up = jnp.clip(u, -LIMIT, LIMIT)
        glu = gate * jax.nn.sigmoid(ALPHA * gate)
        act = (up + 1.0) * glu
        part = jnp.dot(act, wd_ref[0], preferred_element_type=jnp.float32)

        @pl.when(fb == 0)
        def _():
            o_ref[0] = part + bd_ref[0]

        @pl.when(fb != 0)
        def _():
            o_ref[0] = o_ref[0] + part


def _experts(counts, buf, wg, wu, bg, bu, wd, bd, bc=512, bf=512):
    return pl.pallas_call(
        functools.partial(_expert_body, bc=bc),
        grid=(E, C // bc, F // bf),
        in_specs=[
            pl.BlockSpec(memory_space=pltpu.SMEM),
            pl.BlockSpec((1, bc, D), lambda e, cb, fb: (e, cb, 0)),
            pl.BlockSpec((1, D, bf), lambda e, cb, fb: (e, 0, fb)),
            pl.BlockSpec((1, D, bf), lambda e, cb, fb: (e, 0, fb)),
            pl.BlockSpec((1, 1, bf), lambda e, cb, fb: (e, 0, fb)),
            pl.BlockSpec((1, 1, bf), lambda e, cb, fb: (e, 0, fb)),
            pl.BlockSpec((1, bf, D), lambda e, cb, fb: (e, fb, 0)),
            pl.BlockSpec((1, 1, D), lambda e, cb, fb: (e, 0, 0)),
        ],
        out_specs=pl.BlockSpec((1, bc, D), lambda e, cb, fb: (e, cb, 0)),
        out_shape=jax.ShapeDtypeStruct((E, C, D), jnp.float32),
    )(counts, buf, wg, wu, bg, bu, wd, bd)


def _combine_body(x_ref, g_ref, cw_ref, o_ref):
    cw1 = cw_ref[:, 0:1]
    cw2 = cw_ref[:, 1:2]
    g = g_ref[...]
    o_ref[...] = x_ref[...] + cw1 * g[:, :D] + cw2 * g[:, D:]


def _combine(x, gath2, cw, bt=256):
    return pl.pallas_call(
        _combine_body,
        grid=(T // bt,),
        in_specs=[
            pl.BlockSpec((bt, D), lambda t: (t, 0)),
            pl.BlockSpec((bt, K * D), lambda t: (t, 0)),
            pl.BlockSpec((bt, K), lambda t: (t, 0)),
        ],
        out_specs=pl.BlockSpec((bt, D), lambda t: (t, 0)),
        out_shape=jax.ShapeDtypeStruct((T, D), jnp.float32),
    )(x, gath2, cw)


def kernel(hidden_states, positions, ln1_w, w_qkv, b_qkv, sinks, w_o, b_o,
           ln2_w, w_router, b_router, w_gate_up, b_gate_up, w_down, b_down):
    half = HD // 2
    qkv = _rms_matmul(hidden_states, w_qkv, b_qkv, ln1_w)
    q = qkv[:, :HQ * HD].reshape(T, HQ, HD)
    k = qkv[:, HQ * HD:(HQ + HKV) * HD].reshape(T, HKV, HD)
    v = qkv[:, (HQ + HKV) * HD:]

    pos_f = positions.astype(jnp.float32).reshape(T, 1)
    inv = (1.0 / (BASE ** (jnp.arange(half, dtype=jnp.float32) / half))).reshape(1, half)
    q = _rope(q, pos_f, inv, HQ).reshape(T, HQ * HD)
    k = _rope(k, pos_f, inv, HKV).reshape(T, HKV * HD)

    a = _attention(q, k, v, sinks)
    x = _oproj_residual(a, w_o, b_o, hidden_states)
    h2, logits = _rms2_logits(x, ln2_w, w_router, b_router)

    dest, csrc, cw, counts = _route(logits)
    buf = _sc_dispatch(h2, dest.reshape(T * K)).reshape(E, CP, D)

    wg = w_gate_up[:, :, :F]
    wu = w_gate_up[:, :, F:]
    bg = b_gate_up[:, :F].reshape(E, 1, F)
    bu = b_gate_up[:, F:].reshape(E, 1, F)
    y = _experts(counts, buf, wg, wu, bg, bu, w_down, b_down.reshape(E, 1, D))

    gath = _sc_gather(y.reshape(E * C, D), csrc.reshape(T * K))
    return _combine(x, gath.reshape(T, K * D), cw)
